# baseline (speedup 1.0000x reference)
import jax
import jax.numpy as jnp
from jax import lax
from jax.experimental import pallas as pl
from jax.experimental.pallas import tpu as pltpu
from jax.experimental.pallas import tpu_sc as plsc

_CHUNK = 64


def _sc_body(tab_hbm, out_hbm, buf):
    batch, seq_len, embed_dim = out_hbm.shape
    nw = 32
    rows_per_w = seq_len // nw
    wid = lax.axis_index("s") * 2 + lax.axis_index("c")
    base = wid * rows_per_w
    for c in range(rows_per_w // _CHUNK):
        off = base + c * _CHUNK
        pltpu.sync_copy(tab_hbm.at[pl.ds(off, _CHUNK), :], buf)
        for b in range(batch):
            pltpu.sync_copy(buf, out_hbm.at[b, pl.ds(off, _CHUNK), :])


def kernel(x, position_embedding):
    batch, seq_len, embed_dim = x.shape
    mesh = plsc.VectorSubcoreMesh(core_axis_name="c", subcore_axis_name="s")
    k = pl.kernel(
        _sc_body,
        out_type=jax.ShapeDtypeStruct((batch, seq_len, embed_dim),
                                      position_embedding.dtype),
        mesh=mesh,
        scratch_types=[pltpu.VMEM((_CHUNK, embed_dim),
                                  position_embedding.dtype)],
    )
    return k(position_embedding[:seq_len])


# speedup vs baseline: 1.0206x; 1.0206x over previous
import jax
import jax.numpy as jnp
from jax import lax
from jax.experimental import pallas as pl
from jax.experimental.pallas import tpu as pltpu
from jax.experimental.pallas import tpu_sc as plsc

_SC_ROWS = 4096
_SC_WORKERS = 32
_SC_CHUNK = 64
_TC_BLK = 1024


def _sc_body(tab_hbm, out_hbm, buf):
    batch = out_hbm.shape[0]
    rows_per_w = _SC_ROWS // _SC_WORKERS
    wid = lax.axis_index("s") * 2 + lax.axis_index("c")
    base = wid * rows_per_w
    for c in range(rows_per_w // _SC_CHUNK):
        off = base + c * _SC_CHUNK
        pltpu.sync_copy(tab_hbm.at[pl.ds(off, _SC_CHUNK), :], buf)
        for b in range(batch):
            pltpu.sync_copy(buf, out_hbm.at[b, pl.ds(off, _SC_CHUNK), :])


def _tc_body(tab_hbm, part_hbm, out_hbm, buf, rsem, wsem):
    del part_hbm
    batch, seq_len, _ = out_hbm.shape
    n = (seq_len - _SC_ROWS) // _TC_BLK

    def read(i):
        return pltpu.make_async_copy(
            tab_hbm.at[pl.ds(_SC_ROWS + i * _TC_BLK, _TC_BLK), :],
            buf.at[i % 2], rsem.at[i % 2])

    def write(i, b):
        return pltpu.make_async_copy(
            buf.at[i % 2],
            out_hbm.at[b, pl.ds(_SC_ROWS + i * _TC_BLK, _TC_BLK), :],
            wsem.at[i % 2])

    read(0).start()
    for i in range(n):
        read(i).wait()
        for b in range(batch):
            write(i, b).start()
        if i + 1 < n:
            if i >= 1:
                for b in range(batch):
                    write(i - 1, b).wait()
            read(i + 1).start()
    if n >= 2:
        for b in range(batch):
            write(n - 2, b).wait()
    for b in range(batch):
        write(n - 1, b).wait()


def kernel(x, position_embedding):
    batch, seq_len, embed_dim = x.shape
    dtype = position_embedding.dtype
    tab = position_embedding[:seq_len]

    sc_mesh = plsc.VectorSubcoreMesh(core_axis_name="c", subcore_axis_name="s")
    sc_fill = pl.kernel(
        _sc_body,
        out_type=jax.ShapeDtypeStruct((batch, seq_len, embed_dim), dtype),
        mesh=sc_mesh,
        scratch_types=[pltpu.VMEM((_SC_CHUNK, embed_dim), dtype)],
    )
    partial = sc_fill(tab)

    return pl.pallas_call(
        _tc_body,
        in_specs=[pl.BlockSpec(memory_space=pltpu.MemorySpace.HBM),
                  pl.BlockSpec(memory_space=pltpu.MemorySpace.HBM)],
        out_specs=pl.BlockSpec(memory_space=pltpu.MemorySpace.HBM),
        out_shape=jax.ShapeDtypeStruct((batch, seq_len, embed_dim), dtype),
        input_output_aliases={1: 0},
        scratch_shapes=[
            pltpu.VMEM((2, _TC_BLK, embed_dim), dtype),
            pltpu.SemaphoreType.DMA((2,)),
            pltpu.SemaphoreType.DMA((2,)),
        ],
    )(tab, partial)
